# baseline (device time: 2127994 ns/iter reference)
import jax
import jax.numpy as jnp
from jax import lax
from jax.experimental import pallas as pl
from jax.experimental.pallas import tpu as pltpu

C = 16


def kernel(x):
    m_per, n = x.shape
    half = m_per // 2
    rows = half // C

    def body(x_ref, out_ref, copy_sem, y_send, y_recv, x_send, x_recv):
        my_x = lax.axis_index("x")
        my_y = lax.axis_index("y")
        y_nbr = (my_x, 1 - my_y)
        x_nbr = (1 - my_x, my_y)

        barrier_sem = pltpu.get_barrier_semaphore()
        for nbr in (y_nbr, x_nbr):
            pl.semaphore_signal(
                barrier_sem, inc=1, device_id=nbr,
                device_id_type=pl.DeviceIdType.MESH,
            )
        pl.semaphore_wait(barrier_sem, 2)

        local = pltpu.make_async_copy(
            x_ref, out_ref.at[pl.ds(my_y * m_per, m_per)], copy_sem
        )
        local.start()

        mine_base = my_y * m_per
        other_base = (1 - my_y) * m_per
        send_half = my_x * half

        y_sends = []
        for c in range(C):
            r = send_half + c * rows
            d = pltpu.make_async_remote_copy(
                src_ref=x_ref.at[pl.ds(r, rows)],
                dst_ref=out_ref.at[pl.ds(mine_base + r, rows)],
                send_sem=y_send.at[c],
                recv_sem=y_recv.at[c],
                device_id=y_nbr,
                device_id_type=pl.DeviceIdType.MESH,
            )
            d.start()
            y_sends.append(d)

        x_sends = []
        for c in range(C):
            rrow = other_base + send_half + c * rows
            recv_d = pltpu.make_async_remote_copy(
                src_ref=x_ref.at[pl.ds(0, rows)],
                dst_ref=out_ref.at[pl.ds(rrow, rows)],
                send_sem=y_send.at[c],
                recv_sem=y_recv.at[c],
                device_id=y_nbr,
                device_id_type=pl.DeviceIdType.MESH,
            )
            recv_d.wait_recv()
            f = pltpu.make_async_remote_copy(
                src_ref=out_ref.at[pl.ds(rrow, rows)],
                dst_ref=out_ref.at[pl.ds(rrow, rows)],
                send_sem=x_send.at[c],
                recv_sem=x_recv.at[c],
                device_id=x_nbr,
                device_id_type=pl.DeviceIdType.MESH,
            )
            f.start()
            x_sends.append(f)

        for c in range(C):
            rrow = other_base + (1 - my_x) * half + c * rows
            wait_d = pltpu.make_async_remote_copy(
                src_ref=x_ref.at[pl.ds(0, rows)],
                dst_ref=out_ref.at[pl.ds(rrow, rows)],
                send_sem=x_send.at[c],
                recv_sem=x_recv.at[c],
                device_id=x_nbr,
                device_id_type=pl.DeviceIdType.MESH,
            )
            wait_d.wait_recv()

        for d in y_sends:
            d.wait_send()
        for d in x_sends:
            d.wait_send()
        local.wait()

    out_shape = jax.ShapeDtypeStruct((2 * m_per, n), jnp.float32)
    return pl.pallas_call(
        body,
        out_shape=out_shape,
        in_specs=[pl.BlockSpec(memory_space=pl.ANY)],
        out_specs=pl.BlockSpec(memory_space=pl.ANY),
        scratch_shapes=[
            pltpu.SemaphoreType.DMA,
            pltpu.SemaphoreType.DMA((C,)),
            pltpu.SemaphoreType.DMA((C,)),
            pltpu.SemaphoreType.DMA((C,)),
            pltpu.SemaphoreType.DMA((C,)),
        ],
        compiler_params=pltpu.CompilerParams(collective_id=0),
    )(x)


# device time: 476091 ns/iter; 4.4697x vs baseline; 4.4697x over previous
import jax
import jax.numpy as jnp
from jax import lax
from jax.experimental import pallas as pl
from jax.experimental.pallas import tpu as pltpu

C = 16
V_ROWS = 2048


def kernel(x):
    m_per, n = x.shape
    half = m_per // 2
    rows = half // C
    nv = m_per // V_ROWS

    def body(x_ref, out_ref, y_send, y_recv, x_send, x_recv,
             vbuf, ld_sems, st_sems):
        my_x = lax.axis_index("x")
        my_y = lax.axis_index("y")
        y_nbr = (my_x, 1 - my_y)
        x_nbr = (1 - my_x, my_y)

        barrier_sem = pltpu.get_barrier_semaphore()
        for nbr in (y_nbr, x_nbr):
            pl.semaphore_signal(
                barrier_sem, inc=1, device_id=nbr,
                device_id_type=pl.DeviceIdType.MESH,
            )
        pl.semaphore_wait(barrier_sem, 2)

        mine_base = my_y * m_per
        other_base = (1 - my_y) * m_per
        send_half = my_x * half

        y_sends = []
        for c in range(C):
            r = send_half + c * rows
            d = pltpu.make_async_remote_copy(
                src_ref=x_ref.at[pl.ds(r, rows)],
                dst_ref=out_ref.at[pl.ds(mine_base + r, rows)],
                send_sem=y_send.at[c],
                recv_sem=y_recv.at[c],
                device_id=y_nbr,
                device_id_type=pl.DeviceIdType.MESH,
            )
            d.start()
            y_sends.append(d)

        copy_stores = []

        def copy_step(k):
            slot = k % 2
            if k >= 2:
                copy_stores[k - 2].wait()
            ld = pltpu.make_async_copy(
                x_ref.at[pl.ds(k * V_ROWS, V_ROWS)],
                vbuf.at[slot],
                ld_sems.at[slot],
            )
            ld.start()
            ld.wait()
            st = pltpu.make_async_copy(
                vbuf.at[slot],
                out_ref.at[pl.ds(mine_base + k * V_ROWS, V_ROWS)],
                st_sems.at[slot],
            )
            st.start()
            copy_stores.append(st)

        steps_per_iter = (nv + C - 1) // C
        next_copy = 0
        x_sends = []
        for c in range(C):
            rrow = other_base + send_half + c * rows
            recv_d = pltpu.make_async_remote_copy(
                src_ref=x_ref.at[pl.ds(0, rows)],
                dst_ref=out_ref.at[pl.ds(rrow, rows)],
                send_sem=y_send.at[c],
                recv_sem=y_recv.at[c],
                device_id=y_nbr,
                device_id_type=pl.DeviceIdType.MESH,
            )
            recv_d.wait_recv()
            f = pltpu.make_async_remote_copy(
                src_ref=out_ref.at[pl.ds(rrow, rows)],
                dst_ref=out_ref.at[pl.ds(rrow, rows)],
                send_sem=x_send.at[c],
                recv_sem=x_recv.at[c],
                device_id=x_nbr,
                device_id_type=pl.DeviceIdType.MESH,
            )
            f.start()
            x_sends.append(f)
            for _ in range(steps_per_iter):
                if next_copy < nv:
                    copy_step(next_copy)
                    next_copy += 1
        while next_copy < nv:
            copy_step(next_copy)
            next_copy += 1

        for c in range(C):
            rrow = other_base + (1 - my_x) * half + c * rows
            wait_d = pltpu.make_async_remote_copy(
                src_ref=x_ref.at[pl.ds(0, rows)],
                dst_ref=out_ref.at[pl.ds(rrow, rows)],
                send_sem=x_send.at[c],
                recv_sem=x_recv.at[c],
                device_id=x_nbr,
                device_id_type=pl.DeviceIdType.MESH,
            )
            wait_d.wait_recv()

        for d in y_sends:
            d.wait_send()
        for d in x_sends:
            d.wait_send()
        for st in copy_stores[-2:]:
            st.wait()

    out_shape = jax.ShapeDtypeStruct((2 * m_per, n), jnp.float32)
    return pl.pallas_call(
        body,
        out_shape=out_shape,
        in_specs=[pl.BlockSpec(memory_space=pl.ANY)],
        out_specs=pl.BlockSpec(memory_space=pl.ANY),
        scratch_shapes=[
            pltpu.SemaphoreType.DMA((C,)),
            pltpu.SemaphoreType.DMA((C,)),
            pltpu.SemaphoreType.DMA((C,)),
            pltpu.SemaphoreType.DMA((C,)),
            pltpu.VMEM((2, V_ROWS, n), jnp.float32),
            pltpu.SemaphoreType.DMA((2,)),
            pltpu.SemaphoreType.DMA((2,)),
        ],
        compiler_params=pltpu.CompilerParams(collective_id=0),
    )(x)


# device time: 465431 ns/iter; 4.5721x vs baseline; 1.0229x over previous
import jax
import jax.numpy as jnp
from jax import lax
from jax.experimental import pallas as pl
from jax.experimental.pallas import tpu as pltpu

C = 32
V_ROWS = 2048


def kernel(x):
    m_per, n = x.shape
    half = m_per // 2
    rows = half // C
    nv = m_per // V_ROWS

    def body(x_ref, out_ref, y_send, y_recv, x_send, x_recv,
             vbuf, ld_sems, st_sems):
        my_x = lax.axis_index("x")
        my_y = lax.axis_index("y")
        y_nbr = (my_x, 1 - my_y)
        x_nbr = (1 - my_x, my_y)

        barrier_sem = pltpu.get_barrier_semaphore()
        for nbr in (y_nbr, x_nbr):
            pl.semaphore_signal(
                barrier_sem, inc=1, device_id=nbr,
                device_id_type=pl.DeviceIdType.MESH,
            )
        pl.semaphore_wait(barrier_sem, 2)

        mine_base = my_y * m_per
        other_base = (1 - my_y) * m_per
        send_half = my_x * half

        y_sends = []
        for c in range(C):
            r = send_half + c * rows
            d = pltpu.make_async_remote_copy(
                src_ref=x_ref.at[pl.ds(r, rows)],
                dst_ref=out_ref.at[pl.ds(mine_base + r, rows)],
                send_sem=y_send.at[c],
                recv_sem=y_recv.at[c],
                device_id=y_nbr,
                device_id_type=pl.DeviceIdType.MESH,
            )
            d.start()
            y_sends.append(d)

        copy_stores = []

        def copy_step(k):
            slot = k % 2
            if k >= 2:
                copy_stores[k - 2].wait()
            ld = pltpu.make_async_copy(
                x_ref.at[pl.ds(k * V_ROWS, V_ROWS)],
                vbuf.at[slot],
                ld_sems.at[slot],
            )
            ld.start()
            ld.wait()
            st = pltpu.make_async_copy(
                vbuf.at[slot],
                out_ref.at[pl.ds(mine_base + k * V_ROWS, V_ROWS)],
                st_sems.at[slot],
            )
            st.start()
            copy_stores.append(st)

        steps_per_iter = (nv + C - 1) // C
        next_copy = 0
        x_sends = []
        for c in range(C):
            rrow = other_base + send_half + c * rows
            recv_d = pltpu.make_async_remote_copy(
                src_ref=x_ref.at[pl.ds(0, rows)],
                dst_ref=out_ref.at[pl.ds(rrow, rows)],
                send_sem=y_send.at[c],
                recv_sem=y_recv.at[c],
                device_id=y_nbr,
                device_id_type=pl.DeviceIdType.MESH,
            )
            recv_d.wait_recv()
            f = pltpu.make_async_remote_copy(
                src_ref=out_ref.at[pl.ds(rrow, rows)],
                dst_ref=out_ref.at[pl.ds(rrow, rows)],
                send_sem=x_send.at[c],
                recv_sem=x_recv.at[c],
                device_id=x_nbr,
                device_id_type=pl.DeviceIdType.MESH,
            )
            f.start()
            x_sends.append(f)
            for _ in range(steps_per_iter):
                if next_copy < nv:
                    copy_step(next_copy)
                    next_copy += 1
        while next_copy < nv:
            copy_step(next_copy)
            next_copy += 1

        for c in range(C):
            rrow = other_base + (1 - my_x) * half + c * rows
            wait_d = pltpu.make_async_remote_copy(
                src_ref=x_ref.at[pl.ds(0, rows)],
                dst_ref=out_ref.at[pl.ds(rrow, rows)],
                send_sem=x_send.at[c],
                recv_sem=x_recv.at[c],
                device_id=x_nbr,
                device_id_type=pl.DeviceIdType.MESH,
            )
            wait_d.wait_recv()

        for d in y_sends:
            d.wait_send()
        for d in x_sends:
            d.wait_send()
        for st in copy_stores[-2:]:
            st.wait()

    out_shape = jax.ShapeDtypeStruct((2 * m_per, n), jnp.float32)
    return pl.pallas_call(
        body,
        out_shape=out_shape,
        in_specs=[pl.BlockSpec(memory_space=pl.ANY)],
        out_specs=pl.BlockSpec(memory_space=pl.ANY),
        scratch_shapes=[
            pltpu.SemaphoreType.DMA((C,)),
            pltpu.SemaphoreType.DMA((C,)),
            pltpu.SemaphoreType.DMA((C,)),
            pltpu.SemaphoreType.DMA((C,)),
            pltpu.VMEM((2, V_ROWS, n), jnp.float32),
            pltpu.SemaphoreType.DMA((2,)),
            pltpu.SemaphoreType.DMA((2,)),
        ],
        compiler_params=pltpu.CompilerParams(collective_id=0),
    )(x)
